# baseline (device time: 7249 ns/iter reference)
import jax
import jax.numpy as jnp
from jax import lax
from jax.experimental import pallas as pl
from jax.experimental.pallas import tpu as pltpu


def kernel(x):
    m_per, n = x.shape

    def body(
        x_ref,
        out_ref,
        x_vmem,
        bf_send,
        bf_recv,
        f32_stage,
        send_sem,
        recv_sem,
        own_sem,
        stage_sem,
        store_sem,
    ):
        my_x = lax.axis_index("x")
        my_y = lax.axis_index("y")
        my_z = lax.axis_index("z")
        peer = (1 - my_x, my_y, my_z)

        own = pltpu.make_async_copy(
            x_ref, out_ref.at[pl.ds(my_x * m_per, m_per), :], own_sem
        )
        own.start()
        stage = pltpu.make_async_copy(x_ref, x_vmem, stage_sem)
        stage.start()

        barrier_sem = pltpu.get_barrier_semaphore()
        pl.semaphore_signal(
            barrier_sem, inc=1, device_id=peer,
            device_id_type=pl.DeviceIdType.MESH,
        )
        pl.semaphore_wait(barrier_sem, 1)

        stage.wait()
        bf_send[...] = x_vmem[...].astype(jnp.bfloat16)

        rdma = pltpu.make_async_remote_copy(
            src_ref=bf_send,
            dst_ref=bf_recv,
            send_sem=send_sem,
            recv_sem=recv_sem,
            device_id=peer,
            device_id_type=pl.DeviceIdType.MESH,
        )
        rdma.start()
        rdma.wait_recv()

        f32_stage[...] = bf_recv[...].astype(jnp.float32)
        store = pltpu.make_async_copy(
            f32_stage, out_ref.at[pl.ds((1 - my_x) * m_per, m_per), :], store_sem
        )
        store.start()

        own.wait()
        store.wait()
        rdma.wait_send()

    return pl.pallas_call(
        body,
        out_shape=jax.ShapeDtypeStruct((2 * m_per, n), x.dtype),
        in_specs=[pl.BlockSpec(memory_space=pl.ANY)],
        out_specs=pl.BlockSpec(memory_space=pl.ANY),
        scratch_shapes=[
            pltpu.VMEM((m_per, n), jnp.float32),
            pltpu.VMEM((m_per, n), jnp.bfloat16),
            pltpu.VMEM((m_per, n), jnp.bfloat16),
            pltpu.VMEM((m_per, n), jnp.float32),
            pltpu.SemaphoreType.DMA,
            pltpu.SemaphoreType.DMA,
            pltpu.SemaphoreType.DMA,
            pltpu.SemaphoreType.DMA,
            pltpu.SemaphoreType.DMA,
        ],
        compiler_params=pltpu.CompilerParams(collective_id=0),
    )(x)


# device time: 7205 ns/iter; 1.0061x vs baseline; 1.0061x over previous
import jax
import jax.numpy as jnp
from jax import lax
from jax.experimental import pallas as pl
from jax.experimental.pallas import tpu as pltpu


def kernel(x):
    m_per, n = x.shape
    x = pltpu.with_memory_space_constraint(x, pltpu.MemorySpace.HBM)

    def body(
        x_ref,
        out_ref,
        x_vmem,
        bf_send,
        bf_recv,
        send_sem,
        recv_sem,
        own_sem,
        stage_sem,
    ):
        my_x = lax.axis_index("x")
        my_y = lax.axis_index("y")
        my_z = lax.axis_index("z")
        peer = (1 - my_x, my_y, my_z)

        own = pltpu.make_async_copy(
            x_ref, out_ref.at[pl.ds(my_x * m_per, m_per), :], own_sem
        )
        own.start()
        stage = pltpu.make_async_copy(x_ref, x_vmem, stage_sem)
        stage.start()

        barrier_sem = pltpu.get_barrier_semaphore()
        pl.semaphore_signal(
            barrier_sem, inc=1, device_id=peer,
            device_id_type=pl.DeviceIdType.MESH,
        )
        pl.semaphore_wait(barrier_sem, 1)

        stage.wait()
        bf_send[...] = x_vmem[...].astype(jnp.bfloat16)

        rdma = pltpu.make_async_remote_copy(
            src_ref=bf_send,
            dst_ref=bf_recv,
            send_sem=send_sem,
            recv_sem=recv_sem,
            device_id=peer,
            device_id_type=pl.DeviceIdType.MESH,
        )
        rdma.start()
        rdma.wait_recv()

        out_ref[pl.ds((1 - my_x) * m_per, m_per), :] = bf_recv[...].astype(
            jnp.float32
        )

        own.wait()
        rdma.wait_send()

    return pl.pallas_call(
        body,
        out_shape=jax.ShapeDtypeStruct((2 * m_per, n), x.dtype),
        in_specs=[pl.BlockSpec(memory_space=pltpu.MemorySpace.HBM)],
        out_specs=pl.BlockSpec(memory_space=pltpu.VMEM),
        scratch_shapes=[
            pltpu.VMEM((m_per, n), jnp.float32),
            pltpu.VMEM((m_per, n), jnp.bfloat16),
            pltpu.VMEM((m_per, n), jnp.bfloat16),
            pltpu.SemaphoreType.DMA,
            pltpu.SemaphoreType.DMA,
            pltpu.SemaphoreType.DMA,
            pltpu.SemaphoreType.DMA,
        ],
        compiler_params=pltpu.CompilerParams(collective_id=0),
    )(x)


# device time: 7110 ns/iter; 1.0195x vs baseline; 1.0134x over previous
import jax
import jax.numpy as jnp
from jax import lax
from jax.experimental import pallas as pl
from jax.experimental.pallas import tpu as pltpu

N_CHUNKS = 4


def kernel(x):
    m_per, n = x.shape
    x = pltpu.with_memory_space_constraint(x, pltpu.MemorySpace.HBM)
    rows = m_per // N_CHUNKS

    def body(
        x_ref,
        out_ref,
        x_vmem,
        bf_send,
        bf_recv,
        send_sems,
        recv_sems,
        own_sem,
        stage_sems,
    ):
        my_x = lax.axis_index("x")
        my_y = lax.axis_index("y")
        my_z = lax.axis_index("z")
        peer = (1 - my_x, my_y, my_z)

        own = pltpu.make_async_copy(
            x_ref, out_ref.at[pl.ds(my_x * m_per, m_per), :], own_sem
        )
        own.start()
        stages = []
        for c in range(N_CHUNKS):
            sl = pl.ds(c * rows, rows)
            st = pltpu.make_async_copy(
                x_ref.at[sl, :], x_vmem.at[sl, :], stage_sems.at[c]
            )
            st.start()
            stages.append((sl, st))

        barrier_sem = pltpu.get_barrier_semaphore()
        pl.semaphore_signal(
            barrier_sem, inc=1, device_id=peer,
            device_id_type=pl.DeviceIdType.MESH,
        )
        pl.semaphore_wait(barrier_sem, 1)

        rdmas = []
        for sl, st in stages:
            st.wait()
            bf_send[sl, :] = x_vmem[sl, :].astype(jnp.bfloat16)
            rdma = pltpu.make_async_remote_copy(
                src_ref=bf_send.at[sl, :],
                dst_ref=bf_recv.at[sl, :],
                send_sem=send_sems.at[len(rdmas)],
                recv_sem=recv_sems.at[len(rdmas)],
                device_id=peer,
                device_id_type=pl.DeviceIdType.MESH,
            )
            rdma.start()
            rdmas.append((sl, rdma))

        for c, (sl, rdma) in enumerate(rdmas):
            rdma.wait_recv()
            out_sl = pl.ds((1 - my_x) * m_per + c * rows, rows)
            out_ref[out_sl, :] = bf_recv[sl, :].astype(jnp.float32)

        own.wait()
        for sl, rdma in rdmas:
            rdma.wait_send()

    return pl.pallas_call(
        body,
        out_shape=jax.ShapeDtypeStruct((2 * m_per, n), x.dtype),
        in_specs=[pl.BlockSpec(memory_space=pltpu.MemorySpace.HBM)],
        out_specs=pl.BlockSpec(memory_space=pltpu.VMEM),
        scratch_shapes=[
            pltpu.VMEM((m_per, n), jnp.float32),
            pltpu.VMEM((m_per, n), jnp.bfloat16),
            pltpu.VMEM((m_per, n), jnp.bfloat16),
            pltpu.SemaphoreType.DMA((N_CHUNKS,)),
            pltpu.SemaphoreType.DMA((N_CHUNKS,)),
            pltpu.SemaphoreType.DMA,
            pltpu.SemaphoreType.DMA((N_CHUNKS,)),
        ],
        compiler_params=pltpu.CompilerParams(collective_id=0),
    )(x)
